# Initial kernel scaffold; baseline (speedup 1.0000x reference)
import jax
import jax.numpy as jnp
from jax.experimental import pallas as pl


def kernel(x, edge_index, batch, W1_0, b1_0, W2_0, b2_0, gamma_0, beta_0, W1_1, b1_1, W2_1, b2_1, gamma_1, beta_1):
    raise NotImplementedError("write your pallas kernel here")


# speedup vs baseline: 5.1675x; 5.1675x over previous
import jax
import jax.numpy as jnp
from jax import lax
from jax.experimental import pallas as pl
from jax.experimental.pallas import tpu as pltpu
from jax.experimental.pallas import tpu_sc as plsc

N_NODES = 10000
D_FEAT = 128
HIDDEN = 64
NUM_GRAPHS = 64
BN_EPS = 1e-5
N_EDGES = 320000

NC = 2
NS = 16
CHUNK = 128
KCH = -(-N_EDGES // (NC * NS * CHUNK))
E_PAD = NC * NS * KCH * CHUNK
NPAD = 10240
ROWS_PER_TILE = NPAD // NS
ZROWS = 64


def _make_agg_body(width):
    def _agg_body(feat_hbm, src_hbm, dst_hbm, out_hbm,
                  src_v, dst_v, ebuf, zbuf, acc, sem):
        cid = lax.axis_index("c")
        sid = lax.axis_index("s")
        pltpu.sync_copy(src_hbm.at[cid, sid], src_v)
        pltpu.sync_copy(dst_hbm.at[cid, sid], dst_v)
        zero = jnp.zeros((16,), jnp.float32)
        for r in range(ZROWS):
            for c in range(width // 16):
                zbuf[r, pl.ds(c * 16, 16)] = zero
        row0 = sid * ROWS_PER_TILE
        for b in range(ROWS_PER_TILE // ZROWS):
            pltpu.sync_copy(zbuf, acc.at[pl.ds(row0 + b * ZROWS, ZROWS)])
        plsc.subcore_barrier()

        def body(j, carry):
            pltpu.async_copy(feat_hbm.at[src_v.at[j]], ebuf, sem).wait()
            pltpu.sync_copy(ebuf, acc.at[dst_v.at[j]], add=True)
            return carry

        lax.fori_loop(0, KCH, body, 0)
        plsc.subcore_barrier()
        pltpu.sync_copy(acc.at[pl.ds(row0, ROWS_PER_TILE)],
                        out_hbm.at[cid, pl.ds(row0, ROWS_PER_TILE)])

    return _agg_body


def _edge_aggregate(feat, src3, dst3):
    width = feat.shape[1]
    mesh = plsc.VectorSubcoreMesh(core_axis_name="c", subcore_axis_name="s")
    f = pl.kernel(
        _make_agg_body(width),
        out_type=jax.ShapeDtypeStruct((NC, NPAD, width), jnp.float32),
        mesh=mesh,
        scratch_types=[
            pltpu.VMEM((KCH, CHUNK), jnp.int32),
            pltpu.VMEM((KCH, CHUNK), jnp.int32),
            pltpu.VMEM((CHUNK, width), jnp.float32),
            pltpu.VMEM((ZROWS, width), jnp.float32),
            pltpu.VMEM_SHARED((NPAD, width), jnp.float32),
            pltpu.SemaphoreType.DMA,
        ],
        compiler_params=pltpu.CompilerParams(use_tc_tiling_on_sc=False),
    )
    return f(feat, src3, dst3)


def _dot_mxu(a, b):
    return jnp.dot(a.astype(jnp.bfloat16), b.astype(jnp.bfloat16),
                   preferred_element_type=jnp.float32)


def _layer_tail(x, p0, p1, w1, b1, w2, b2, g, be):
    pre = x + (p0 + p1)
    t = _dot_mxu(pre, w1) + b1
    h = jnp.maximum(t, 0.0)
    t2 = _dot_mxu(h, w2) + b2
    mean = jnp.mean(t2, axis=0, keepdims=True)
    var = jnp.mean((t2 - mean) ** 2, axis=0, keepdims=True)
    y = g * (t2 - mean) / jnp.sqrt(var + BN_EPS) + be
    return jnp.maximum(y, 0.0)


def _layer0_body(x_ref, parts_ref, w1_ref, b1_ref, w2_ref, b2_ref,
                 g_ref, be_ref, o_ref):
    o_ref[...] = _layer_tail(
        x_ref[...], parts_ref[0, :N_NODES, :], parts_ref[1, :N_NODES, :],
        w1_ref[...], b1_ref[...], w2_ref[...], b2_ref[...],
        g_ref[...], be_ref[...])


def _layer1_body(h1_ref, parts_ref, w1_ref, b1_ref, w2_ref, b2_ref,
                 g_ref, be_ref, batch_ref, h_ref, hg_ref):
    h2 = _layer_tail(
        h1_ref[...], parts_ref[0, :N_NODES, :], parts_ref[1, :N_NODES, :],
        w1_ref[...], b1_ref[...], w2_ref[...], b2_ref[...],
        g_ref[...], be_ref[...])
    h_ref[...] = h2
    b = batch_ref[...]
    gidx = lax.broadcasted_iota(jnp.int32, (NUM_GRAPHS, NPAD), 0)
    seg = (gidx == b).astype(jnp.bfloat16)
    h2p = jnp.concatenate(
        [h2, jnp.zeros((NPAD - N_NODES, HIDDEN), jnp.float32)], axis=0)
    hi = h2p.astype(jnp.bfloat16)
    lo = (h2p - hi.astype(jnp.float32)).astype(jnp.bfloat16)
    sums = (jnp.dot(seg, hi, preferred_element_type=jnp.float32)
            + jnp.dot(seg, lo, preferred_element_type=jnp.float32))
    cnt = jnp.sum(seg.astype(jnp.float32), axis=1, keepdims=True)
    hg_ref[...] = sums / jnp.maximum(cnt, 1.0)


def kernel(x, edge_index, batch,
           W1_0, b1_0, W2_0, b2_0, gamma_0, beta_0,
           W1_1, b1_1, W2_1, b2_1, gamma_1, beta_1):
    f32 = jnp.float32
    src = edge_index[0].astype(jnp.int32)
    dst = edge_index[1].astype(jnp.int32)
    pad = E_PAD - N_EDGES
    src3 = jnp.concatenate([src, jnp.zeros((pad,), jnp.int32)]
                           ).reshape(NC, NS, KCH, CHUNK)
    dst3 = jnp.concatenate([dst, jnp.full((pad,), N_NODES, jnp.int32)]
                           ).reshape(NC, NS, KCH, CHUNK)
    batch_pad = jnp.concatenate(
        [batch.astype(jnp.int32),
         jnp.full((NPAD - N_NODES,), NUM_GRAPHS, jnp.int32)]).reshape(1, NPAD)

    parts0 = _edge_aggregate(x, src3, dst3)
    h1 = pl.pallas_call(
        _layer0_body,
        out_shape=jax.ShapeDtypeStruct((N_NODES, HIDDEN), f32))(
        x, parts0, W1_0, b1_0.reshape(1, HIDDEN), W2_0,
        b2_0.reshape(1, HIDDEN), gamma_0.reshape(1, HIDDEN),
        beta_0.reshape(1, HIDDEN))
    parts1 = _edge_aggregate(h1, src3, dst3)
    h, hg = pl.pallas_call(
        _layer1_body,
        out_shape=(jax.ShapeDtypeStruct((N_NODES, HIDDEN), f32),
                   jax.ShapeDtypeStruct((NUM_GRAPHS, HIDDEN), f32)))(
        h1, parts1, W1_1, b1_1.reshape(1, HIDDEN), W2_1,
        b2_1.reshape(1, HIDDEN), gamma_1.reshape(1, HIDDEN),
        beta_1.reshape(1, HIDDEN), batch_pad)
    return (h, hg)
